# baseline (speedup 1.0000x reference)
import jax
import jax.numpy as jnp
from jax import lax
from jax.experimental import pallas as pl
from jax.experimental.pallas import tpu as pltpu
from jax.experimental.pallas import tpu_sc as plsc

_P = 0.2
_B = 100000
_F = 128
_NC = 2
_NS = 16
_NW = _NC * _NS
_RPW = _B // _NW
_RCH = 125
_NCH = _RPW // _RCH
_KW = _F // 4

_XC = _RCH * _F
_KC = _RCH * _KW


def _sc_body(x_hbm, kw_hbm, ti_hbm, tw_hbm, out_hbm,
             xb0, xb1, kb0, kb1, ob0, ob1, tiv, twv,
             semi0, semi1, semo0, semo1):
    wid = lax.axis_index("s") * _NC + lax.axis_index("c")
    base = wid * (_RPW * _F)
    kbase = wid * (_RPW * _KW)

    pltpu.sync_copy(ti_hbm, tiv)
    pltpu.sync_copy(tw_hbm, twv)

    iota = lax.iota(jnp.int32, 16)
    q = lax.shift_right_logical(iota, 2)
    bitm = lax.shift_left(jnp.full((16,), 1, jnp.int32),
                          lax.shift_left(jnp.bitwise_and(iota, 3), 3))

    i0 = [tiv[pl.ds(g * 16, 16)] for g in range(8)]
    i1 = [tiv[pl.ds(_F + g * 16, 16)] for g in range(8)]
    w0 = [twv[pl.ds(g * 16, 16)] for g in range(8)]
    w1 = [twv[pl.ds(_F + g * 16, 16)] for g in range(8)]
    kq = [q + (g * 4) for g in range(8)]

    xbufs = (xb0, xb1)
    kbufs = (kb0, kb1)
    obufs = (ob0, ob1)
    isems = (semi0, semi1)
    osems = (semo0, semo1)

    scale_keep = jnp.float32(1.0 / (1.0 - _P))

    def issue_in(t, s):
        hx = pltpu.async_copy(
            x_hbm.at[pl.ds(base + t * _XC, _XC)], xbufs[s], isems[s])
        hk = pltpu.async_copy(
            kw_hbm.at[pl.ds(kbase + t * _KC, _KC)], kbufs[s], isems[s])
        return (hx, hk)

    def compute(s):
        xb = xbufs[s]
        kb = kbufs[s]
        ob = obufs[s]

        def row(r, _):
            rx = jnp.full((16,), 0, jnp.int32) + r * _F
            rk = jnp.full((16,), 0, jnp.int32) + r * _KW
            for g in range(8):
                xv = xb[pl.ds(r * _F + g * 16, 16)]
                g0 = plsc.load_gather(xb, [rx + i0[g]])
                g1 = plsc.load_gather(xb, [rx + i1[g]])
                wt = g0 * w0[g] + g1 * w1[g]
                kw = plsc.load_gather(kb, [rk + kq[g]])
                keepv = jnp.bitwise_and(kw, bitm) != 0
                ob[pl.ds(r * _F + g * 16, 16)] = jnp.where(
                    keepv, xv * scale_keep, wt)
            return 0

        lax.fori_loop(0, _RCH, row, 0)

    def issue_out(t, s):
        return pltpu.async_copy(
            obufs[s], out_hbm.at[pl.ds(base + t * _XC, _XC)], osems[s])

    in_h = [None, None]
    out_h = [None, None]
    in_h[0] = issue_in(0, 0)
    for t in range(_NCH):
        s = t % 2
        if t + 1 < _NCH:
            in_h[1 - s] = issue_in(t + 1, 1 - s)
        hx, hk = in_h[s]
        hx.wait()
        hk.wait()
        if out_h[s] is not None:
            out_h[s].wait()
        compute(s)
        out_h[s] = issue_out(t, s)
    out_h[0].wait()
    out_h[1].wait()


def kernel(x, m_w, m_idx, keep):
    kwords = lax.bitcast_convert_type(
        keep.astype(jnp.uint8).reshape(_B, _KW, 4), jnp.int32)

    midx = m_idx.astype(jnp.int32)
    inv_p = jnp.float32(1.0 / (_P + 1e-12))
    ti = jnp.concatenate([midx[:, 0], midx[:, 1]])
    tw = jnp.concatenate([m_w[:, 0] * inv_p, m_w[:, 1] * inv_p])

    mesh = plsc.VectorSubcoreMesh(core_axis_name="c", subcore_axis_name="s")
    out_flat = pl.kernel(
        _sc_body,
        out_type=jax.ShapeDtypeStruct((_B * _F,), jnp.float32),
        mesh=mesh,
        compiler_params=pltpu.CompilerParams(needs_layout_passes=False),
        scratch_types=[
            pltpu.VMEM((_XC,), jnp.float32),
            pltpu.VMEM((_XC,), jnp.float32),
            pltpu.VMEM((_KC,), jnp.int32),
            pltpu.VMEM((_KC,), jnp.int32),
            pltpu.VMEM((_XC,), jnp.float32),
            pltpu.VMEM((_XC,), jnp.float32),
            pltpu.VMEM((_F * 2,), jnp.int32),
            pltpu.VMEM((_F * 2,), jnp.float32),
            pltpu.SemaphoreType.DMA,
            pltpu.SemaphoreType.DMA,
            pltpu.SemaphoreType.DMA,
            pltpu.SemaphoreType.DMA,
        ],
    )(x.reshape(-1), kwords.reshape(-1), ti, tw)
    return out_flat.reshape(_B, _F)


# speedup vs baseline: 1.8959x; 1.8959x over previous
import jax
import jax.numpy as jnp
from jax import lax
from jax.experimental import pallas as pl
from jax.experimental.pallas import tpu as pltpu
from jax.experimental.pallas import tpu_sc as plsc

_P = 0.2
_B = 100000
_F = 128
_NC = 2
_NS = 16
_NW = _NC * _NS
_RCH = 160
_NCHT = _B // _RCH
_NCHW = 20


def _sc_body(x_hbm, k_hbm, ti_hbm, tw_hbm, out_hbm,
             xb0, xb1, kb0, kb1, cb0, cb1, tiv, twv,
             semi0, semi1, semo0, semo1):
    wid = lax.axis_index("s") * _NC + lax.axis_index("c")

    pltpu.sync_copy(ti_hbm, tiv)
    pltpu.sync_copy(tw_hbm, twv)

    inf = jnp.float32(jnp.inf)
    zero = jnp.float32(0.0)
    scale_keep = jnp.float32(1.0 / (1.0 - _P))

    xbufs = (xb0, xb1)
    kbufs = (kb0, kb1)
    cbufs = (cb0, cb1)
    isems = (semi0, semi1)
    osems = (semo0, semo1)

    def chunk_row(t):
        c = jnp.minimum(t * _NW + wid, _NCHT - 1)
        return pl.multiple_of(c * _RCH, 32)

    def issue_in(t, s):
        r = chunk_row(t)
        hx = pltpu.async_copy(x_hbm.at[pl.ds(r, _RCH), :], xbufs[s], isems[s])
        hk = pltpu.async_copy(k_hbm.at[pl.ds(r, _RCH), :], kbufs[s], isems[s])
        return (hx, hk)

    def issue_out(t, s):
        r = chunk_row(t)
        return pltpu.async_copy(cbufs[s], out_hbm.at[pl.ds(r, _RCH), :],
                                osems[s])

    def compute(s):
        xb = xbufs[s]
        kb = kbufs[s]
        cb = cbufs[s]

        def row_a(r, _):
            for g in range(8):
                kf = kb[r, pl.ds(g * 16, 16)]
                xv = xb[r, pl.ds(g * 16, 16)]
                cb[r, pl.ds(g * 16, 16)] = jnp.where(
                    kf != zero, xv * scale_keep, inf)
            return 0

        lax.fori_loop(0, _RCH, row_a, 0, unroll=False)

        def group_b(g, _):
            off = pl.multiple_of(g * 16, 16)
            i0g = tiv[pl.ds(off, 16)]
            i1g = tiv[pl.ds(off + _F, 16)]
            w0g = twv[pl.ds(off, 16)]
            w1g = twv[pl.ds(off + _F, 16)]

            def row_b(r, _):
                rx = jnp.full((16,), 0, jnp.int32) + r
                g0 = plsc.load_gather(xb, [rx, i0g])
                g1 = plsc.load_gather(xb, [rx, i1g])
                wt = g0 * w0g + g1 * w1g
                c = cb[r, pl.ds(off, 16)]
                cb[r, pl.ds(off, 16)] = jnp.where(c == inf, wt, c)
                return 0

            lax.fori_loop(0, _RCH, row_b, 0, unroll=False)
            return 0

        lax.fori_loop(0, 8, group_b, 0, unroll=False)

    in_h = [None, None]
    out_h = [None, None]
    in_h[0] = issue_in(0, 0)
    for t in range(_NCHW):
        s = t % 2
        if t + 1 < _NCHW:
            in_h[1 - s] = issue_in(t + 1, 1 - s)
        hx, hk = in_h[s]
        hx.wait()
        hk.wait()
        if out_h[s] is not None:
            out_h[s].wait()
        compute(s)
        out_h[s] = issue_out(t, s)
    out_h[0].wait()
    out_h[1].wait()


def kernel(x, m_w, m_idx, keep):
    kf32 = keep.astype(jnp.float32)

    midx = m_idx.astype(jnp.int32)
    inv_p = jnp.float32(1.0 / (_P + 1e-12))
    ti = jnp.concatenate([midx[:, 0], midx[:, 1]])
    tw = jnp.concatenate([m_w[:, 0] * inv_p, m_w[:, 1] * inv_p])

    mesh = plsc.VectorSubcoreMesh(core_axis_name="c", subcore_axis_name="s")
    out = pl.kernel(
        _sc_body,
        out_type=jax.ShapeDtypeStruct((_B, _F), jnp.float32),
        mesh=mesh,
        compiler_params=pltpu.CompilerParams(needs_layout_passes=False),
        scratch_types=[
            pltpu.VMEM((_RCH, _F), jnp.float32),
            pltpu.VMEM((_RCH, _F), jnp.float32),
            pltpu.VMEM((_RCH, _F), jnp.float32),
            pltpu.VMEM((_RCH, _F), jnp.float32),
            pltpu.VMEM((_RCH, _F), jnp.float32),
            pltpu.VMEM((_RCH, _F), jnp.float32),
            pltpu.VMEM((_F * 2,), jnp.int32),
            pltpu.VMEM((_F * 2,), jnp.float32),
            pltpu.SemaphoreType.DMA,
            pltpu.SemaphoreType.DMA,
            pltpu.SemaphoreType.DMA,
            pltpu.SemaphoreType.DMA,
        ],
    )(x, kf32, ti, tw)
    return out


# speedup vs baseline: 2.1118x; 1.1139x over previous
import jax
import jax.numpy as jnp
from jax import lax
from jax.experimental import pallas as pl
from jax.experimental.pallas import tpu as pltpu
from jax.experimental.pallas import tpu_sc as plsc

_P = 0.2
_B = 100000
_F = 128
_NC = 2
_NS = 16
_NW = _NC * _NS
_RCH = 160
_NCHT = _B // _RCH
_NCHW = 20


def _sc_body(x_hbm, k_hbm, ti_hbm, tw_hbm, out_hbm,
             xb0, xb1, kb0, kb1, cb0, cb1, tiv, twv,
             semi0, semi1, semo0, semo1):
    wid = lax.axis_index("s") * _NC + lax.axis_index("c")

    pltpu.sync_copy(ti_hbm, tiv)
    pltpu.sync_copy(tw_hbm, twv)

    zero = jnp.float32(0.0)
    scale_keep = jnp.float32(1.0 / (1.0 - _P))

    xbufs = (xb0, xb1)
    kbufs = (kb0, kb1)
    cbufs = (cb0, cb1)
    isems = (semi0, semi1)
    osems = (semo0, semo1)

    def chunk_row(t):
        c = jnp.minimum(t * _NW + wid, _NCHT - 1)
        return pl.multiple_of(c * _RCH, 32)

    def issue_in(t, s):
        r = chunk_row(t)
        hx = pltpu.async_copy(x_hbm.at[pl.ds(r, _RCH), :], xbufs[s], isems[s])
        hk = pltpu.async_copy(k_hbm.at[pl.ds(r, _RCH), :], kbufs[s], isems[s])
        return (hx, hk)

    def issue_out(t, s):
        r = chunk_row(t)
        return pltpu.async_copy(cbufs[s], out_hbm.at[pl.ds(r, _RCH), :],
                                osems[s])

    def compute(s):
        xb = xbufs[s]
        kb = kbufs[s]
        cb = cbufs[s]

        def group_b(g, _):
            off = pl.multiple_of(g * 16, 16)
            i0g = tiv[pl.ds(off, 16)]
            i1g = tiv[pl.ds(off + _F, 16)]
            w0g = twv[pl.ds(off, 16)]
            w1g = twv[pl.ds(off + _F, 16)]

            def row_b(r, _):
                rx = jnp.full((16,), 0, jnp.int32) + r
                g0 = plsc.load_gather(xb, [rx, i0g])
                g1 = plsc.load_gather(xb, [rx, i1g])
                wt = g0 * w0g + g1 * w1g
                kf = kb[r, pl.ds(off, 16)]
                xv = xb[r, pl.ds(off, 16)]
                cb[r, pl.ds(off, 16)] = jnp.where(
                    kf != zero, xv * scale_keep, wt)
                return 0

            lax.fori_loop(0, _RCH, row_b, 0, unroll=False)
            return 0

        lax.fori_loop(0, 8, group_b, 0, unroll=False)

    in_h = [None, None]
    out_h = [None, None]
    in_h[0] = issue_in(0, 0)
    for t in range(_NCHW):
        s = t % 2
        if t + 1 < _NCHW:
            in_h[1 - s] = issue_in(t + 1, 1 - s)
        hx, hk = in_h[s]
        hx.wait()
        hk.wait()
        if out_h[s] is not None:
            out_h[s].wait()
        compute(s)
        out_h[s] = issue_out(t, s)
    out_h[0].wait()
    out_h[1].wait()


def kernel(x, m_w, m_idx, keep):
    kf32 = keep.astype(jnp.float32)

    midx = m_idx.astype(jnp.int32)
    inv_p = jnp.float32(1.0 / (_P + 1e-12))
    ti = jnp.concatenate([midx[:, 0], midx[:, 1]])
    tw = jnp.concatenate([m_w[:, 0] * inv_p, m_w[:, 1] * inv_p])

    mesh = plsc.VectorSubcoreMesh(core_axis_name="c", subcore_axis_name="s")
    out = pl.kernel(
        _sc_body,
        out_type=jax.ShapeDtypeStruct((_B, _F), jnp.float32),
        mesh=mesh,
        compiler_params=pltpu.CompilerParams(needs_layout_passes=False),
        scratch_types=[
            pltpu.VMEM((_RCH, _F), jnp.float32),
            pltpu.VMEM((_RCH, _F), jnp.float32),
            pltpu.VMEM((_RCH, _F), jnp.float32),
            pltpu.VMEM((_RCH, _F), jnp.float32),
            pltpu.VMEM((_RCH, _F), jnp.float32),
            pltpu.VMEM((_RCH, _F), jnp.float32),
            pltpu.VMEM((_F * 2,), jnp.int32),
            pltpu.VMEM((_F * 2,), jnp.float32),
            pltpu.SemaphoreType.DMA,
            pltpu.SemaphoreType.DMA,
            pltpu.SemaphoreType.DMA,
            pltpu.SemaphoreType.DMA,
        ],
    )(x, kf32, ti, tw)
    return out


# speedup vs baseline: 2.1325x; 1.0098x over previous
import jax
import jax.numpy as jnp
from jax import lax
from jax.experimental import pallas as pl
from jax.experimental.pallas import tpu as pltpu
from jax.experimental.pallas import tpu_sc as plsc

_P = 0.2
_B = 100000
_F = 128
_NC = 2
_NS = 16
_NW = _NC * _NS
_RCH = 160
_NCHT = _B // _RCH
_NCHW = 20


def _sc_body(x_hbm, k_hbm, ti_hbm, tw_hbm, out_hbm,
             xb0, xb1, kb0, kb1, cb0, cb1, tiv, twv,
             semi0, semi1, semo0, semo1):
    wid = lax.axis_index("s") * _NC + lax.axis_index("c")

    pltpu.sync_copy(ti_hbm, tiv)
    pltpu.sync_copy(tw_hbm, twv)

    zero = jnp.float32(0.0)
    scale_keep = jnp.float32(1.0 / (1.0 - _P))

    xbufs = (xb0, xb1)
    kbufs = (kb0, kb1)
    cbufs = (cb0, cb1)
    isems = (semi0, semi1)
    osems = (semo0, semo1)

    def chunk_row(t):
        c = jnp.minimum(t * _NW + wid, _NCHT - 1)
        return pl.multiple_of(c * _RCH, 32)

    def issue_in(t, s):
        r = chunk_row(t)
        hx = pltpu.async_copy(x_hbm.at[pl.ds(r, _RCH), :], xbufs[s], isems[s])
        hk = pltpu.async_copy(k_hbm.at[pl.ds(r, _RCH), :], kbufs[s], isems[s])
        return (hx, hk)

    def issue_out(t, s):
        r = chunk_row(t)
        return pltpu.async_copy(cbufs[s], out_hbm.at[pl.ds(r, _RCH), :],
                                osems[s])

    def compute(s):
        xb = xbufs[s]
        kb = kbufs[s]
        cb = cbufs[s]

        def group_b(g, _):
            off = pl.multiple_of(g * 16, 16)
            i0g = tiv[pl.ds(off, 16)]
            i1g = tiv[pl.ds(off + _F, 16)]
            w0g = twv[pl.ds(off, 16)]
            w1g = twv[pl.ds(off + _F, 16)]

            def row_b(r, _):
                rx = jnp.full((16,), 0, jnp.int32) + r
                g0 = plsc.load_gather(xb, [rx, i0g])
                g1 = plsc.load_gather(xb, [rx, i1g])
                wt = g0 * w0g + g1 * w1g
                kf = kb[r, pl.ds(off, 16)]
                xv = xb[r, pl.ds(off, 16)]
                cb[r, pl.ds(off, 16)] = jnp.where(
                    kf != zero, xv * scale_keep, wt)
                return 0

            lax.fori_loop(0, _RCH, row_b, 0, unroll=4)
            return 0

        lax.fori_loop(0, 8, group_b, 0, unroll=False)

    in_h = [None, None]
    out_h = [None, None]
    in_h[0] = issue_in(0, 0)
    for t in range(_NCHW):
        s = t % 2
        if t + 1 < _NCHW:
            in_h[1 - s] = issue_in(t + 1, 1 - s)
        hx, hk = in_h[s]
        hx.wait()
        hk.wait()
        if out_h[s] is not None:
            out_h[s].wait()
        compute(s)
        out_h[s] = issue_out(t, s)
    out_h[0].wait()
    out_h[1].wait()


def kernel(x, m_w, m_idx, keep):
    kf32 = keep.astype(jnp.float32)

    midx = m_idx.astype(jnp.int32)
    inv_p = jnp.float32(1.0 / (_P + 1e-12))
    ti = jnp.concatenate([midx[:, 0], midx[:, 1]])
    tw = jnp.concatenate([m_w[:, 0] * inv_p, m_w[:, 1] * inv_p])

    mesh = plsc.VectorSubcoreMesh(core_axis_name="c", subcore_axis_name="s")
    out = pl.kernel(
        _sc_body,
        out_type=jax.ShapeDtypeStruct((_B, _F), jnp.float32),
        mesh=mesh,
        compiler_params=pltpu.CompilerParams(needs_layout_passes=False),
        scratch_types=[
            pltpu.VMEM((_RCH, _F), jnp.float32),
            pltpu.VMEM((_RCH, _F), jnp.float32),
            pltpu.VMEM((_RCH, _F), jnp.float32),
            pltpu.VMEM((_RCH, _F), jnp.float32),
            pltpu.VMEM((_RCH, _F), jnp.float32),
            pltpu.VMEM((_RCH, _F), jnp.float32),
            pltpu.VMEM((_F * 2,), jnp.int32),
            pltpu.VMEM((_F * 2,), jnp.float32),
            pltpu.SemaphoreType.DMA,
            pltpu.SemaphoreType.DMA,
            pltpu.SemaphoreType.DMA,
            pltpu.SemaphoreType.DMA,
        ],
    )(x, kf32, ti, tw)
    return out
